# baseline (speedup 1.0000x reference)
import jax
import jax.numpy as jnp
from jax.experimental import pallas as pl
from jax.experimental.pallas import tpu as pltpu

_N = 10000
_F_IN = 128
_F_OUT = 32
_NUM_CLASSES = 10
_HALF = _N // 2


def _fused_gcn_cell(x_ref, wz_ref, bz_ref, wh_ref, bh_ref, wl_ref, bl_ref,
                    o_ref):
    az = (wz_ref[0, 0, :_F_IN, :] + wz_ref[1, 0, :_F_IN, :]) * -0.5
    ah = wh_ref[0, 0, :_F_IN, :] + wh_ref[1, 0, :_F_IN, :]
    zero64 = jnp.zeros((_F_IN, 2 * _F_OUT), jnp.float32)
    w_lo = jnp.concatenate([az, ah, zero64], axis=1)
    w_hi = jnp.concatenate([zero64, az, ah], axis=1)
    bcat = jnp.concatenate(
        [bz_ref[...] * -0.5, bh_ref[...]] * 2, axis=1)

    g = (jnp.dot(x_ref[:_HALF, :], w_lo, preferred_element_type=jnp.float32)
         + jnp.dot(x_ref[_HALF:, :], w_hi,
                   preferred_element_type=jnp.float32)
         + bcat)
    t = jnp.tanh(g)
    t_sh = pltpu.roll(t, 3 * _F_OUT, axis=1)
    h = jax.nn.relu((1.0 + t) * t_sh)
    wl = wl_ref[...] * 0.5
    zero_wl = jnp.zeros((_F_OUT, _NUM_CLASSES), jnp.float32)
    wl2 = jnp.concatenate(
        [jnp.concatenate([wl, zero_wl], axis=1),
         jnp.concatenate([zero_wl, zero_wl], axis=1),
         jnp.concatenate([zero_wl, wl], axis=1),
         jnp.concatenate([zero_wl, zero_wl], axis=1)], axis=0)
    p = jnp.dot(h, wl2, preferred_element_type=jnp.float32) + \
        jnp.concatenate([bl_ref[...]] * 2, axis=1)
    o_ref[:_HALF, :] = p[:, :_NUM_CLASSES]
    o_ref[_HALF:, :] = p[:, _NUM_CLASSES:]


def kernel(x, edge_index, edge_weight, Wz, bz, Wr, br, Wh, bh, Wl, bl):
    del edge_index, edge_weight, Wr, br
    return pl.pallas_call(
        _fused_gcn_cell,
        out_shape=jax.ShapeDtypeStruct((_N, _NUM_CLASSES), jnp.float32),
    )(x, Wz, bz.reshape(1, _F_OUT), Wh, bh.reshape(1, _F_OUT), Wl,
      bl.reshape(1, _NUM_CLASSES))
